# baseline (speedup 1.0000x reference)
import functools

import jax
import jax.numpy as jnp
from jax import lax
from jax.experimental import pallas as pl
from jax.experimental.pallas import tpu as pltpu
from jax.experimental.pallas import tpu_sc as plsc

N = 10000
D_IN = 128
D_H = 512
N_CLASS = 47
N_HEADS = 4

_BLK = 1000

_SC_NC = 2
_SC_NS = 16
_NW = _SC_NC * _SC_NS
_EB = 128
_BPW = 80
_EP = _NW * _BPW * _EB
_NROWS = 10112
_ZROW = _NROWS // _SC_NS


_GB = 8
_NG = _BPW // _GB


def _segsum_multi_kernel(nchunks):
    def body(*refs):
        hn_refs = refs[:nchunks]
        src_hbm, dst_hbm, zr_hbm = refs[nchunks:nchunks + 3]
        out_refs = refs[nchunks + 3:2 * nchunks + 3]
        (idxsA, idxdA, idxsB, idxdB, buf0, buf1, agg,
         semiA, semiB, sem0, sem1) = refs[2 * nchunks + 3:]
        c = lax.axis_index("c")
        s = lax.axis_index("s")
        wid = c * _SC_NS + s
        base = wid * _BPW
        bufs = (buf0, buf1)
        sems = (sem0, sem1)

        def process_group(hn_hbm, idxs, idxd):
            pltpu.async_copy(hn_hbm.at[idxs.at[0]], bufs[0], sems[0])
            for b in range(_GB):
                if b + 1 < _GB:
                    pltpu.async_copy(hn_hbm.at[idxs.at[b + 1]],
                                     bufs[(b + 1) % 2], sems[(b + 1) % 2])
                pltpu.make_async_copy(hn_hbm.at[idxs.at[b]],
                                      bufs[b % 2], sems[b % 2]).wait()
                pltpu.sync_copy(bufs[b % 2], agg.at[idxd.at[b]], add=True)

        def idx_start(g, idxs, idxd, sem):
            row = base + g * _GB
            pltpu.async_copy(src_hbm.at[pl.ds(row, _GB)], idxs, sem)
            pltpu.async_copy(dst_hbm.at[pl.ds(row, _GB)], idxd, sem)

        def idx_wait(g, idxs, idxd, sem):
            row = base + g * _GB
            pltpu.make_async_copy(src_hbm.at[pl.ds(row, _GB)], idxs, sem).wait()
            pltpu.make_async_copy(dst_hbm.at[pl.ds(row, _GB)], idxd, sem).wait()

        for chunk in range(nchunks):
            hn_hbm = hn_refs[chunk]
            out_hbm = out_refs[chunk]
            pltpu.sync_copy(zr_hbm, agg.at[pl.ds(s * _ZROW, _ZROW)])
            plsc.subcore_barrier()

            pltpu.sync_copy(src_hbm.at[pl.ds(base, _GB)], idxsA)
            pltpu.sync_copy(dst_hbm.at[pl.ds(base, _GB)], idxdA)
            idx_start(1, idxsB, idxdB, semiB)

            def pair(k, carry):
                g = 2 * k
                process_group(hn_hbm, idxsA, idxdA)
                idx_wait(g + 1, idxsB, idxdB, semiB)

                @pl.when(k < _NG // 2 - 1)
                def _():
                    idx_start(g + 2, idxsA, idxdA, semiA)

                process_group(hn_hbm, idxsB, idxdB)

                @pl.when(k < _NG // 2 - 1)
                def _():
                    idx_wait(g + 2, idxsA, idxdA, semiA)
                    idx_start(g + 3, idxsB, idxdB, semiB)

                return carry

            lax.fori_loop(0, _NG // 2, pair, 0)
            plsc.subcore_barrier()
            pltpu.sync_copy(agg.at[pl.ds(s * _ZROW, _ZROW)],
                            out_hbm.at[c, pl.ds(s * _ZROW, _ZROW)])

    return body


def _segsum_sc(hns, srcr, dstr, zeros):
    nchunks = len(hns)
    mesh = plsc.VectorSubcoreMesh(core_axis_name="c", subcore_axis_name="s")
    out = pl.kernel(
        _segsum_multi_kernel(nchunks),
        out_type=[jax.ShapeDtypeStruct((_SC_NC, _NROWS, D_IN), jnp.float32)
                  for _ in range(nchunks)],
        mesh=mesh,
        scratch_types=[
            pltpu.VMEM((_GB, _EB), jnp.int32),
            pltpu.VMEM((_GB, _EB), jnp.int32),
            pltpu.VMEM((_GB, _EB), jnp.int32),
            pltpu.VMEM((_GB, _EB), jnp.int32),
            pltpu.VMEM((_EB, D_IN), jnp.float32),
            pltpu.VMEM((_EB, D_IN), jnp.float32),
            pltpu.VMEM_SHARED((_NROWS, D_IN), jnp.float32),
            pltpu.SemaphoreType.DMA,
            pltpu.SemaphoreType.DMA,
            pltpu.SemaphoreType.DMA,
            pltpu.SemaphoreType.DMA,
        ],
    )(*hns, srcr, dstr, zeros)
    return list(out)


_DROWS = 10240
_DZ = _DROWS // _SC_NS


def _deg_sc_kernel(srcd_hbm, dstd_hbm, ones_hbm, zr_hbm,
                   outs_hbm, outd_hbm, idx, ones_v, acc, sem0):
    c = lax.axis_index("c")
    s = lax.axis_index("s")
    wid = c * _SC_NS + s
    base = wid * _BPW
    pltpu.sync_copy(ones_hbm, ones_v)
    for idx_hbm, out_hbm in ((srcd_hbm, outs_hbm), (dstd_hbm, outd_hbm)):
        pltpu.sync_copy(zr_hbm, acc.at[pl.ds(s * _DZ, _DZ)])
        plsc.subcore_barrier()

        def group(g, carry):
            row = base + g * _GB
            pltpu.sync_copy(idx_hbm.at[pl.ds(row, _GB)], idx)
            for b in range(_GB):
                pltpu.sync_copy(ones_v, acc.at[idx.at[b]], add=True)
            return carry

        lax.fori_loop(0, _NG, group, 0)
        plsc.subcore_barrier()
        pltpu.sync_copy(acc.at[pl.ds(s * _DZ, _DZ)],
                        out_hbm.at[c, pl.ds(s * _DZ, _DZ)])
        plsc.subcore_barrier()


def _degrees_sc(srcd, dstd):
    mesh = plsc.VectorSubcoreMesh(core_axis_name="c", subcore_axis_name="s")
    ones = jnp.ones((_EB, D_IN), jnp.float32)
    zeros = jnp.zeros((_DZ, D_IN), jnp.float32)
    return pl.kernel(
        _deg_sc_kernel,
        out_type=[jax.ShapeDtypeStruct((_SC_NC, _DROWS, D_IN), jnp.float32),
                  jax.ShapeDtypeStruct((_SC_NC, _DROWS, D_IN), jnp.float32)],
        mesh=mesh,
        scratch_types=[
            pltpu.VMEM((_GB, _EB), jnp.int32),
            pltpu.VMEM((_EB, D_IN), jnp.float32),
            pltpu.VMEM_SHARED((_DROWS, D_IN), jnp.float32),
            pltpu.SemaphoreType.DMA,
        ],
    )(srcd, dstd, ones, zeros)


def _pad_edges_deg(src, dst):
    npad = _EP - src.shape[0]
    pad = jnp.arange(npad, dtype=jnp.int32) % (_DROWS - N)
    srcd = jnp.concatenate([src.astype(jnp.int32), N + pad]).reshape(-1, _EB)
    dstd = jnp.concatenate([dst.astype(jnp.int32), N + pad]).reshape(-1, _EB)
    return srcd, dstd


def _pad_edges(src, dst):
    npad = _EP - src.shape[0]
    pad_src = (jnp.arange(npad, dtype=jnp.int32) % 16)
    pad_dst = N + (jnp.arange(npad, dtype=jnp.int32) % (_NROWS - N))
    srcr = jnp.concatenate([src.astype(jnp.int32), pad_src]).reshape(-1, _EB)
    dstr = jnp.concatenate([dst.astype(jnp.int32), pad_dst]).reshape(-1, _EB)
    return srcr, dstr


def _ln(x, g, b, eps=1e-12):
    u = x.mean(-1, keepdims=True)
    s = ((x - u) ** 2).mean(-1, keepdims=True)
    return g * (x - u) / jnp.sqrt(s + eps) + b


def _fm_kernel(feats_ref, labels_ref, table_ref, w1a_ref, w1b_ref, b1_ref,
               g1_ref, be1_ref, w2_ref, b2_ref, out_ref):
    lab_ids = labels_ref[0, 0, :]
    onehot = (lab_ids[:, None] ==
              jax.lax.broadcasted_iota(jnp.int32, (_BLK, 64), 1)).astype(jnp.float32)
    lab = jnp.dot(onehot, table_ref[...], preferred_element_type=jnp.float32)
    x = (jnp.dot(lab, w1a_ref[...], preferred_element_type=jnp.float32)
         + jnp.dot(feats_ref[...], w1b_ref[...], preferred_element_type=jnp.float32)
         + b1_ref[...])
    x = _ln(x, g1_ref[...], be1_ref[...])
    x = jnp.maximum(x, 0.0)
    out_ref[...] = (jnp.dot(x, w2_ref[...], preferred_element_type=jnp.float32)
                    + b2_ref[...])


def _fm_stage(input_feats, input_labels, p):
    table = jnp.zeros((64, D_IN), jnp.float32).at[:N_CLASS + 1].set(p['label_table'])
    w1a = p['fm_W1'][:D_IN]
    w1b = p['fm_W1'][D_IN:]
    labels3d = input_labels.astype(jnp.int32).reshape(N // _BLK, 1, _BLK)
    grid = N // _BLK
    full = lambda shape: pl.BlockSpec(shape, lambda i: tuple(0 for _ in shape))
    return pl.pallas_call(
        _fm_kernel,
        grid=(grid,),
        in_specs=[
            pl.BlockSpec((_BLK, D_IN), lambda i: (i, 0)),
            pl.BlockSpec((1, 1, _BLK), lambda i: (i, 0, 0)),
            full((64, D_IN)),
            full((D_IN, D_H)),
            full((D_IN, D_H)),
            full((D_H,)),
            full((D_H,)),
            full((D_H,)),
            full((D_H, D_IN)),
            full((D_IN,)),
        ],
        out_specs=pl.BlockSpec((_BLK, D_IN), lambda i: (i, 0)),
        out_shape=jax.ShapeDtypeStruct((N, D_IN), jnp.float32),
    )(input_feats, labels3d, table, w1a, w1b, p['fm_b1'], p['fm_g1'],
      p['fm_be1'], p['fm_W2'], p['fm_b2'])


def _dot(a, b):
    return jnp.dot(a.astype(jnp.bfloat16), b.astype(jnp.bfloat16),
                   preferred_element_type=jnp.float32)


def _elu(x):
    return jnp.where(x > 0, x, jnp.exp(x) - 1.0)


def _attn_first_row(branches, Wq, bq, Wk, bk, Wv, bv):
    dh = D_H // N_HEADS
    q0 = _dot(branches[0], Wq) + bq
    ks = [_dot(t, Wk) + bk for t in branches]
    vs = [_dot(t, Wv) + bv for t in branches]
    inv = 1.0 / jnp.sqrt(jnp.float32(dh))
    parts = []
    for hd in range(N_HEADS):
        sl = slice(hd * dh, (hd + 1) * dh)
        qh = q0[:, sl]
        ss = [jnp.sum(qh * k[:, sl], axis=1, keepdims=True) * inv for k in ks]
        m = ss[0]
        for s_ in ss[1:]:
            m = jnp.maximum(m, s_)
        es = [jnp.exp(s_ - m) for s_ in ss]
        den = es[0]
        for e_ in es[1:]:
            den = den + e_
        ctx_h = es[0] / den * vs[0][:, sl]
        for e_, v_ in zip(es[1:], vs[1:]):
            ctx_h = ctx_h + e_ / den * v_[:, sl]
        parts.append(ctx_h)
    return jnp.concatenate(parts, axis=1)


def _layer0_kernel(feat_ref, aggA_ref, aggB_ref, nd_ref, ns_ref,
                   ws_ref, bs_ref, gr_ref, br_ref,
                   wc_ref, bc_ref, gc_ref, bcc_ref,
                   wq_ref, bq_ref, wk_ref, bk_ref, wv_ref, bv_ref,
                   go_ref, bo_ref,
                   hout_ref, hn0_ref, hn1_ref, hn2_ref, hn3_ref):
    h = feat_ref[...]
    h1 = _elu(_ln(_dot(h, ws_ref[...]) + bs_ref[...], gr_ref[...], br_ref[...]))
    nd = nd_ref[0, 0, :][:, None]
    agg = (aggA_ref[0] + aggB_ref[0]) * nd
    h3 = _elu(_ln(_dot(agg, wc_ref[...]) + bc_ref[...], gc_ref[...], bcc_ref[...]))
    ctx = _attn_first_row([h1, h3], wq_ref[...], bq_ref[...], wk_ref[...],
                          bk_ref[...], wv_ref[...], bv_ref[...])
    hout = _ln(ctx, go_ref[...], bo_ref[...])
    hout_ref[...] = hout
    ns = ns_ref[0, 0, :][:, None]
    hn = hout * ns
    for c, ref in enumerate((hn0_ref, hn1_ref, hn2_ref, hn3_ref)):
        ref[...] = hn[:, c * D_IN:(c + 1) * D_IN]


def _layer1_kernel(h_ref, a0A_ref, a0B_ref, a1A_ref, a1B_ref, a2A_ref,
                   a2B_ref, a3A_ref, a3B_ref, feat_ref, nd_ref,
                   ws_ref, bs_ref, gr_ref, br_ref,
                   wc_ref, bc_ref, gc_ref, bcc_ref,
                   wsc_ref, bsc_ref, gsc_ref, bscc_ref,
                   wq_ref, bq_ref, wk_ref, bk_ref, wv_ref, bv_ref,
                   go_ref, bo_ref,
                   hw1_ref, hb1_ref, hg_ref, hbe_ref, hw2_ref, hb2_ref,
                   out_ref):
    h = h_ref[...]
    h1 = _elu(_ln(_dot(h, ws_ref[...]) + bs_ref[...], gr_ref[...], br_ref[...]))
    nd = nd_ref[0, 0, :][:, None]
    agg = jnp.concatenate(
        [a0A_ref[0] + a0B_ref[0], a1A_ref[0] + a1B_ref[0],
         a2A_ref[0] + a2B_ref[0], a3A_ref[0] + a3B_ref[0]], axis=1) * nd
    h3 = _elu(_ln(_dot(agg, wc_ref[...]) + bc_ref[...], gc_ref[...], bcc_ref[...]))
    h5 = _ln(_dot(feat_ref[...], wsc_ref[...]) + bsc_ref[...],
             gsc_ref[...], bscc_ref[...])
    ctx = _attn_first_row([h1, h3, h5], wq_ref[...], bq_ref[...], wk_ref[...],
                          bk_ref[...], wv_ref[...], bv_ref[...])
    hh = _ln(ctx, go_ref[...], bo_ref[...])
    x = _ln(_dot(hh, hw1_ref[...]) + hb1_ref[...], hg_ref[...], hbe_ref[...])
    x = jnp.maximum(x, 0.0)
    out_ref[...] = _dot(x, hw2_ref[...]) + hb2_ref[...]


def _full(shape):
    return pl.BlockSpec(shape, lambda i, _s=len(shape): tuple(0 for _ in range(_s)))


def _blk(shape):
    return pl.BlockSpec(shape, lambda i: (i,) + tuple(0 for _ in shape[1:]))


def _part_specs():
    return [pl.BlockSpec((1, _BLK, D_IN), lambda i: (0, i, 0)),
            pl.BlockSpec((1, _BLK, D_IN), lambda i: (1, i, 0))]


def _vec3(x):
    return x.reshape(N // _BLK, 1, _BLK)


_V3SPEC = pl.BlockSpec((1, 1, _BLK), lambda i: (i, 0, 0))


def _layer0_stage(feature, part0, nd3, ns3, p):
    grid = N // _BLK
    in_specs = ([_blk((_BLK, D_IN))] + _part_specs() + [_V3SPEC, _V3SPEC]
                + [_full(s) for s in
                   [(D_IN, D_H), (D_H,), (D_H,), (D_H,),
                    (D_IN, D_H), (D_H,), (D_H,), (D_H,),
                    (D_H, D_H), (D_H,), (D_H, D_H), (D_H,), (D_H, D_H), (D_H,),
                    (D_H,), (D_H,)]])
    out_specs = [_blk((_BLK, D_H))] + [_blk((_BLK, D_IN))] * 4
    out_shape = ([jax.ShapeDtypeStruct((N, D_H), jnp.float32)]
                 + [jax.ShapeDtypeStruct((N, D_IN), jnp.float32)] * 4)
    return pl.pallas_call(
        _layer0_kernel, grid=(grid,), in_specs=in_specs,
        out_specs=out_specs, out_shape=out_shape,
    )(feature, part0, part0, nd3, ns3,
      p['skip0_W'], p['skip0_b'], p['ln_r0_g'], p['ln_r0_b'],
      p['conv0_W'], p['conv0_b'], p['ln_c0_g'], p['ln_c0_b'],
      p['attn0_Wq'], p['attn0_bq'], p['attn0_Wk'], p['attn0_bk'],
      p['attn0_Wv'], p['attn0_bv'], p['ln_o0_g'], p['ln_o0_b'])


def _layer1_stage(h, parts, feature, nd3, p):
    grid = N // _BLK
    head_W2p = jnp.zeros((D_H, D_IN), jnp.float32).at[:, :N_CLASS].set(p['head_W2'])
    head_b2p = jnp.zeros((D_IN,), jnp.float32).at[:N_CLASS].set(p['head_b2'])
    in_specs = ([_blk((_BLK, D_H))]
                + _part_specs() + _part_specs() + _part_specs() + _part_specs()
                + [_blk((_BLK, D_IN)), _V3SPEC]
                + [_full(s) for s in
                   [(D_H, D_H), (D_H,), (D_H,), (D_H,),
                    (D_H, D_H), (D_H,), (D_H,), (D_H,),
                    (D_IN, D_H), (D_H,), (D_H,), (D_H,),
                    (D_H, D_H), (D_H,), (D_H, D_H), (D_H,), (D_H, D_H), (D_H,),
                    (D_H,), (D_H,),
                    (D_H, D_H), (D_H,), (D_H,), (D_H,), (D_H, D_IN), (D_IN,)]])
    return pl.pallas_call(
        _layer1_kernel, grid=(grid,), in_specs=in_specs,
        out_specs=_blk((_BLK, D_IN)),
        out_shape=jax.ShapeDtypeStruct((N, D_IN), jnp.float32),
    )(h, parts[0], parts[0], parts[1], parts[1], parts[2], parts[2],
      parts[3], parts[3], feature, nd3,
      p['skip1_W'], p['skip1_b'], p['ln_r1_g'], p['ln_r1_b'],
      p['conv1_W'], p['conv1_b'], p['ln_c1_g'], p['ln_c1_b'],
      p['sc_W'], p['sc_b'], p['sc_g'], p['sc_be'],
      p['attn1_Wq'], p['attn1_bq'], p['attn1_Wk'], p['attn1_bk'],
      p['attn1_Wv'], p['attn1_bv'], p['ln_o1_g'], p['ln_o1_b'],
      p['head_W1'], p['head_b1'], p['head_g'], p['head_be'],
      head_W2p, head_b2p)


def kernel(input_feats, params, edge_index, input_labels):
    p = params
    src, dst = edge_index[0], edge_index[1]
    srcd, dstd = _pad_edges_deg(src, dst)
    degs, degd = _degrees_sc(srcd, dstd)
    deg_out = degs[0, :N, 0] + degs[1, :N, 0]
    deg_in = degd[0, :N, 0] + degd[1, :N, 0]
    ns = jnp.clip(deg_out, 1.0) ** -0.5
    nd = jnp.clip(deg_in, 1.0) ** -0.5
    ns3, nd3 = _vec3(ns), _vec3(nd)

    srcr, dstr = _pad_edges(src, dst)
    zeros = jnp.zeros((_ZROW, D_IN), jnp.float32)

    feature = _fm_stage(input_feats, input_labels, p)
    hn0 = feature * ns[:, None]
    part0, = _segsum_sc([hn0], srcr, dstr, zeros)
    h, hn1_0, hn1_1, hn1_2, hn1_3 = _layer0_stage(feature, part0, nd3, ns3, p)
    parts1 = _segsum_sc([hn1_0, hn1_1, hn1_2, hn1_3], srcr, dstr, zeros)
    logits_pad = _layer1_stage(h, parts1, feature, nd3, p)
    return logits_pad[:, :N_CLASS]


# speedup vs baseline: 1.0036x; 1.0036x over previous
import functools

import jax
import jax.numpy as jnp
from jax import lax
from jax.experimental import pallas as pl
from jax.experimental.pallas import tpu as pltpu
from jax.experimental.pallas import tpu_sc as plsc

N = 10000
D_IN = 128
D_H = 512
N_CLASS = 47
N_HEADS = 4

_BLK = 1000

_SC_NC = 2
_SC_NS = 16
_NW = _SC_NC * _SC_NS
_EB = 128
_BPW = 80
_EP = _NW * _BPW * _EB
_NROWS = 10112
_ZROW = _NROWS // _SC_NS


_GB = 8
_NG = _BPW // _GB


def _segsum_multi_kernel(nchunks):
    def body(*refs):
        hn_refs = refs[:nchunks]
        src_hbm, dst_hbm, zr_hbm = refs[nchunks:nchunks + 3]
        out_refs = refs[nchunks + 3:2 * nchunks + 3]
        (idxsA, idxdA, idxsB, idxdB, buf0, buf1, agg,
         semiA, semiB, sem0, sem1) = refs[2 * nchunks + 3:]
        c = lax.axis_index("c")
        s = lax.axis_index("s")
        wid = c * _SC_NS + s
        base = wid * _BPW
        bufs = (buf0, buf1)
        sems = (sem0, sem1)

        def process_group(hn_hbm, idxs, idxd):
            pltpu.async_copy(hn_hbm.at[idxs.at[0]], bufs[0], sems[0])
            for b in range(_GB):
                if b + 1 < _GB:
                    pltpu.async_copy(hn_hbm.at[idxs.at[b + 1]],
                                     bufs[(b + 1) % 2], sems[(b + 1) % 2])
                pltpu.make_async_copy(hn_hbm.at[idxs.at[b]],
                                      bufs[b % 2], sems[b % 2]).wait()
                pltpu.sync_copy(bufs[b % 2], agg.at[idxd.at[b]], add=True)

        def idx_start(g, idxs, idxd, sem):
            row = base + g * _GB
            pltpu.async_copy(src_hbm.at[pl.ds(row, _GB)], idxs, sem)
            pltpu.async_copy(dst_hbm.at[pl.ds(row, _GB)], idxd, sem)

        def idx_wait(g, idxs, idxd, sem):
            row = base + g * _GB
            pltpu.make_async_copy(src_hbm.at[pl.ds(row, _GB)], idxs, sem).wait()
            pltpu.make_async_copy(dst_hbm.at[pl.ds(row, _GB)], idxd, sem).wait()

        for chunk in range(nchunks):
            hn_hbm = hn_refs[chunk]
            out_hbm = out_refs[chunk]
            pltpu.sync_copy(zr_hbm, agg.at[pl.ds(s * _ZROW, _ZROW)])
            plsc.subcore_barrier()

            pltpu.sync_copy(src_hbm.at[pl.ds(base, _GB)], idxsA)
            pltpu.sync_copy(dst_hbm.at[pl.ds(base, _GB)], idxdA)
            idx_start(1, idxsB, idxdB, semiB)

            def pair(k, carry):
                g = 2 * k
                process_group(hn_hbm, idxsA, idxdA)
                idx_wait(g + 1, idxsB, idxdB, semiB)

                @pl.when(k < _NG // 2 - 1)
                def _():
                    idx_start(g + 2, idxsA, idxdA, semiA)

                process_group(hn_hbm, idxsB, idxdB)

                @pl.when(k < _NG // 2 - 1)
                def _():
                    idx_wait(g + 2, idxsA, idxdA, semiA)
                    idx_start(g + 3, idxsB, idxdB, semiB)

                return carry

            lax.fori_loop(0, _NG // 2, pair, 0)
            plsc.subcore_barrier()
            pltpu.sync_copy(agg.at[pl.ds(s * _ZROW, _ZROW)],
                            out_hbm.at[c, pl.ds(s * _ZROW, _ZROW)])

    return body


def _segsum_sc(hns, srcr, dstr, zeros):
    nchunks = len(hns)
    mesh = plsc.VectorSubcoreMesh(core_axis_name="c", subcore_axis_name="s")
    out = pl.kernel(
        _segsum_multi_kernel(nchunks),
        out_type=[jax.ShapeDtypeStruct((_SC_NC, _NROWS, D_IN), jnp.float32)
                  for _ in range(nchunks)],
        mesh=mesh,
        scratch_types=[
            pltpu.VMEM((_GB, _EB), jnp.int32),
            pltpu.VMEM((_GB, _EB), jnp.int32),
            pltpu.VMEM((_GB, _EB), jnp.int32),
            pltpu.VMEM((_GB, _EB), jnp.int32),
            pltpu.VMEM((_EB, D_IN), jnp.float32),
            pltpu.VMEM((_EB, D_IN), jnp.float32),
            pltpu.VMEM_SHARED((_NROWS, D_IN), jnp.float32),
            pltpu.SemaphoreType.DMA,
            pltpu.SemaphoreType.DMA,
            pltpu.SemaphoreType.DMA,
            pltpu.SemaphoreType.DMA,
        ],
    )(*hns, srcr, dstr, zeros)
    return list(out)


_DROWS = 10240
_DZ = _DROWS // _SC_NS


def _deg_sc_kernel(srcd_hbm, dstd_hbm, ones_hbm, zr_hbm,
                   outs_hbm, outd_hbm, idx, ones_v, acc, sem0):
    c = lax.axis_index("c")
    s = lax.axis_index("s")
    wid = c * _SC_NS + s
    base = wid * _BPW
    pltpu.sync_copy(ones_hbm, ones_v)
    for idx_hbm, out_hbm in ((srcd_hbm, outs_hbm), (dstd_hbm, outd_hbm)):
        pltpu.sync_copy(zr_hbm, acc.at[pl.ds(s * _DZ, _DZ)])
        plsc.subcore_barrier()

        def group(g, carry):
            row = base + g * _GB
            pltpu.sync_copy(idx_hbm.at[pl.ds(row, _GB)], idx)
            for b in range(_GB):
                pltpu.sync_copy(ones_v, acc.at[idx.at[b]], add=True)
            return carry

        lax.fori_loop(0, _NG, group, 0)
        plsc.subcore_barrier()
        pltpu.sync_copy(acc.at[pl.ds(s * _DZ, _DZ)],
                        out_hbm.at[c, pl.ds(s * _DZ, _DZ)])
        plsc.subcore_barrier()


def _degrees_sc(srcd, dstd):
    mesh = plsc.VectorSubcoreMesh(core_axis_name="c", subcore_axis_name="s")
    ones = jnp.ones((_EB, D_IN), jnp.float32)
    zeros = jnp.zeros((_DZ, D_IN), jnp.float32)
    return pl.kernel(
        _deg_sc_kernel,
        out_type=[jax.ShapeDtypeStruct((_SC_NC, _DROWS, D_IN), jnp.float32),
                  jax.ShapeDtypeStruct((_SC_NC, _DROWS, D_IN), jnp.float32)],
        mesh=mesh,
        scratch_types=[
            pltpu.VMEM((_GB, _EB), jnp.int32),
            pltpu.VMEM((_EB, D_IN), jnp.float32),
            pltpu.VMEM_SHARED((_DROWS, D_IN), jnp.float32),
            pltpu.SemaphoreType.DMA,
        ],
    )(srcd, dstd, ones, zeros)


def _pad_edges_deg(src, dst):
    npad = _EP - src.shape[0]
    pad = jnp.arange(npad, dtype=jnp.int32) % (_DROWS - N)
    srcd = jnp.concatenate([src.astype(jnp.int32), N + pad]).reshape(-1, _EB)
    dstd = jnp.concatenate([dst.astype(jnp.int32), N + pad]).reshape(-1, _EB)
    return srcd, dstd


def _pad_edges(src, dst):
    npad = _EP - src.shape[0]
    pad_src = (jnp.arange(npad, dtype=jnp.int32) % 16)
    pad_dst = N + (jnp.arange(npad, dtype=jnp.int32) % (_NROWS - N))
    srcr = jnp.concatenate([src.astype(jnp.int32), pad_src]).reshape(-1, _EB)
    dstr = jnp.concatenate([dst.astype(jnp.int32), pad_dst]).reshape(-1, _EB)
    return srcr, dstr


def _ln(x, g, b, eps=1e-12):
    u = x.mean(-1, keepdims=True)
    s = ((x - u) ** 2).mean(-1, keepdims=True)
    return g * (x - u) / jnp.sqrt(s + eps) + b


def _fm_kernel(feats_ref, labels_ref, table_ref, w1a_ref, w1b_ref, b1_ref,
               g1_ref, be1_ref, w2_ref, b2_ref, out_ref):
    lab_ids = labels_ref[0, 0, :]
    onehot = (lab_ids[:, None] ==
              jax.lax.broadcasted_iota(jnp.int32, (_BLK, 64), 1)).astype(jnp.float32)
    lab = jnp.dot(onehot, table_ref[...], preferred_element_type=jnp.float32)
    x = (jnp.dot(lab, w1a_ref[...], preferred_element_type=jnp.float32)
         + jnp.dot(feats_ref[...], w1b_ref[...], preferred_element_type=jnp.float32)
         + b1_ref[...])
    x = _ln(x, g1_ref[...], be1_ref[...])
    x = jnp.maximum(x, 0.0)
    out_ref[...] = (jnp.dot(x, w2_ref[...], preferred_element_type=jnp.float32)
                    + b2_ref[...])


def _fm_stage(input_feats, input_labels, p):
    table = jnp.zeros((64, D_IN), jnp.float32).at[:N_CLASS + 1].set(p['label_table'])
    w1a = p['fm_W1'][:D_IN]
    w1b = p['fm_W1'][D_IN:]
    labels3d = input_labels.astype(jnp.int32).reshape(N // _BLK, 1, _BLK)
    grid = N // _BLK
    full = lambda shape: pl.BlockSpec(shape, lambda i: tuple(0 for _ in shape))
    return pl.pallas_call(
        _fm_kernel,
        grid=(grid,),
        in_specs=[
            pl.BlockSpec((_BLK, D_IN), lambda i: (i, 0)),
            pl.BlockSpec((1, 1, _BLK), lambda i: (i, 0, 0)),
            full((64, D_IN)),
            full((D_IN, D_H)),
            full((D_IN, D_H)),
            full((D_H,)),
            full((D_H,)),
            full((D_H,)),
            full((D_H, D_IN)),
            full((D_IN,)),
        ],
        out_specs=pl.BlockSpec((_BLK, D_IN), lambda i: (i, 0)),
        out_shape=jax.ShapeDtypeStruct((N, D_IN), jnp.float32),
    )(input_feats, labels3d, table, w1a, w1b, p['fm_b1'], p['fm_g1'],
      p['fm_be1'], p['fm_W2'], p['fm_b2'])


def _dot(a, b):
    return jnp.dot(a, b, preferred_element_type=jnp.float32)


def _elu(x):
    return jnp.where(x > 0, x, jnp.exp(x) - 1.0)


def _attn_first_row(branches, Wq, bq, Wk, bk, Wv, bv):
    dh = D_H // N_HEADS
    q0 = _dot(branches[0], Wq) + bq
    ks = [_dot(t, Wk) + bk for t in branches]
    vs = [_dot(t, Wv) + bv for t in branches]
    inv = 1.0 / jnp.sqrt(jnp.float32(dh))
    parts = []
    for hd in range(N_HEADS):
        sl = slice(hd * dh, (hd + 1) * dh)
        qh = q0[:, sl]
        ss = [jnp.sum(qh * k[:, sl], axis=1, keepdims=True) * inv for k in ks]
        m = ss[0]
        for s_ in ss[1:]:
            m = jnp.maximum(m, s_)
        es = [jnp.exp(s_ - m) for s_ in ss]
        den = es[0]
        for e_ in es[1:]:
            den = den + e_
        ctx_h = es[0] / den * vs[0][:, sl]
        for e_, v_ in zip(es[1:], vs[1:]):
            ctx_h = ctx_h + e_ / den * v_[:, sl]
        parts.append(ctx_h)
    return jnp.concatenate(parts, axis=1)


def _layer0_kernel(feat_ref, aggA_ref, aggB_ref, nd_ref, ns_ref,
                   ws_ref, bs_ref, gr_ref, br_ref,
                   wc_ref, bc_ref, gc_ref, bcc_ref,
                   wq_ref, bq_ref, wk_ref, bk_ref, wv_ref, bv_ref,
                   go_ref, bo_ref,
                   hout_ref, hn0_ref, hn1_ref, hn2_ref, hn3_ref):
    h = feat_ref[...]
    h1 = _elu(_ln(_dot(h, ws_ref[...]) + bs_ref[...], gr_ref[...], br_ref[...]))
    nd = nd_ref[0, 0, :][:, None]
    agg = (aggA_ref[0] + aggB_ref[0]) * nd
    h3 = _elu(_ln(_dot(agg, wc_ref[...]) + bc_ref[...], gc_ref[...], bcc_ref[...]))
    ctx = _attn_first_row([h1, h3], wq_ref[...], bq_ref[...], wk_ref[...],
                          bk_ref[...], wv_ref[...], bv_ref[...])
    hout = _ln(ctx, go_ref[...], bo_ref[...])
    hout_ref[...] = hout
    ns = ns_ref[0, 0, :][:, None]
    hn = hout * ns
    for c, ref in enumerate((hn0_ref, hn1_ref, hn2_ref, hn3_ref)):
        ref[...] = hn[:, c * D_IN:(c + 1) * D_IN]


def _layer1_kernel(h_ref, a0A_ref, a0B_ref, a1A_ref, a1B_ref, a2A_ref,
                   a2B_ref, a3A_ref, a3B_ref, feat_ref, nd_ref,
                   ws_ref, bs_ref, gr_ref, br_ref,
                   wc_ref, bc_ref, gc_ref, bcc_ref,
                   wsc_ref, bsc_ref, gsc_ref, bscc_ref,
                   wq_ref, bq_ref, wk_ref, bk_ref, wv_ref, bv_ref,
                   go_ref, bo_ref,
                   hw1_ref, hb1_ref, hg_ref, hbe_ref, hw2_ref, hb2_ref,
                   out_ref):
    h = h_ref[...]
    h1 = _elu(_ln(_dot(h, ws_ref[...]) + bs_ref[...], gr_ref[...], br_ref[...]))
    nd = nd_ref[0, 0, :][:, None]
    agg = jnp.concatenate(
        [a0A_ref[0] + a0B_ref[0], a1A_ref[0] + a1B_ref[0],
         a2A_ref[0] + a2B_ref[0], a3A_ref[0] + a3B_ref[0]], axis=1) * nd
    h3 = _elu(_ln(_dot(agg, wc_ref[...]) + bc_ref[...], gc_ref[...], bcc_ref[...]))
    h5 = _ln(_dot(feat_ref[...], wsc_ref[...]) + bsc_ref[...],
             gsc_ref[...], bscc_ref[...])
    ctx = _attn_first_row([h1, h3, h5], wq_ref[...], bq_ref[...], wk_ref[...],
                          bk_ref[...], wv_ref[...], bv_ref[...])
    hh = _ln(ctx, go_ref[...], bo_ref[...])
    x = _ln(_dot(hh, hw1_ref[...]) + hb1_ref[...], hg_ref[...], hbe_ref[...])
    x = jnp.maximum(x, 0.0)
    out_ref[...] = _dot(x, hw2_ref[...]) + hb2_ref[...]


def _full(shape):
    return pl.BlockSpec(shape, lambda i, _s=len(shape): tuple(0 for _ in range(_s)))


def _blk(shape):
    return pl.BlockSpec(shape, lambda i: (i,) + tuple(0 for _ in shape[1:]))


def _part_specs():
    return [pl.BlockSpec((1, _BLK, D_IN), lambda i: (0, i, 0)),
            pl.BlockSpec((1, _BLK, D_IN), lambda i: (1, i, 0))]


def _vec3(x):
    return x.reshape(N // _BLK, 1, _BLK)


_V3SPEC = pl.BlockSpec((1, 1, _BLK), lambda i: (i, 0, 0))


def _layer0_stage(feature, part0, nd3, ns3, p):
    grid = N // _BLK
    in_specs = ([_blk((_BLK, D_IN))] + _part_specs() + [_V3SPEC, _V3SPEC]
                + [_full(s) for s in
                   [(D_IN, D_H), (D_H,), (D_H,), (D_H,),
                    (D_IN, D_H), (D_H,), (D_H,), (D_H,),
                    (D_H, D_H), (D_H,), (D_H, D_H), (D_H,), (D_H, D_H), (D_H,),
                    (D_H,), (D_H,)]])
    out_specs = [_blk((_BLK, D_H))] + [_blk((_BLK, D_IN))] * 4
    out_shape = ([jax.ShapeDtypeStruct((N, D_H), jnp.float32)]
                 + [jax.ShapeDtypeStruct((N, D_IN), jnp.float32)] * 4)
    return pl.pallas_call(
        _layer0_kernel, grid=(grid,), in_specs=in_specs,
        out_specs=out_specs, out_shape=out_shape,
    )(feature, part0, part0, nd3, ns3,
      p['skip0_W'], p['skip0_b'], p['ln_r0_g'], p['ln_r0_b'],
      p['conv0_W'], p['conv0_b'], p['ln_c0_g'], p['ln_c0_b'],
      p['attn0_Wq'], p['attn0_bq'], p['attn0_Wk'], p['attn0_bk'],
      p['attn0_Wv'], p['attn0_bv'], p['ln_o0_g'], p['ln_o0_b'])


def _layer1_stage(h, parts, feature, nd3, p):
    grid = N // _BLK
    head_W2p = jnp.zeros((D_H, D_IN), jnp.float32).at[:, :N_CLASS].set(p['head_W2'])
    head_b2p = jnp.zeros((D_IN,), jnp.float32).at[:N_CLASS].set(p['head_b2'])
    in_specs = ([_blk((_BLK, D_H))]
                + _part_specs() + _part_specs() + _part_specs() + _part_specs()
                + [_blk((_BLK, D_IN)), _V3SPEC]
                + [_full(s) for s in
                   [(D_H, D_H), (D_H,), (D_H,), (D_H,),
                    (D_H, D_H), (D_H,), (D_H,), (D_H,),
                    (D_IN, D_H), (D_H,), (D_H,), (D_H,),
                    (D_H, D_H), (D_H,), (D_H, D_H), (D_H,), (D_H, D_H), (D_H,),
                    (D_H,), (D_H,),
                    (D_H, D_H), (D_H,), (D_H,), (D_H,), (D_H, D_IN), (D_IN,)]])
    return pl.pallas_call(
        _layer1_kernel, grid=(grid,), in_specs=in_specs,
        out_specs=_blk((_BLK, D_IN)),
        out_shape=jax.ShapeDtypeStruct((N, D_IN), jnp.float32),
    )(h, parts[0], parts[0], parts[1], parts[1], parts[2], parts[2],
      parts[3], parts[3], feature, nd3,
      p['skip1_W'], p['skip1_b'], p['ln_r1_g'], p['ln_r1_b'],
      p['conv1_W'], p['conv1_b'], p['ln_c1_g'], p['ln_c1_b'],
      p['sc_W'], p['sc_b'], p['sc_g'], p['sc_be'],
      p['attn1_Wq'], p['attn1_bq'], p['attn1_Wk'], p['attn1_bk'],
      p['attn1_Wv'], p['attn1_bv'], p['ln_o1_g'], p['ln_o1_b'],
      p['head_W1'], p['head_b1'], p['head_g'], p['head_be'],
      head_W2p, head_b2p)


def kernel(input_feats, params, edge_index, input_labels):
    p = params
    src, dst = edge_index[0], edge_index[1]
    srcd, dstd = _pad_edges_deg(src, dst)
    degs, degd = _degrees_sc(srcd, dstd)
    deg_out = degs[0, :N, 0] + degs[1, :N, 0]
    deg_in = degd[0, :N, 0] + degd[1, :N, 0]
    ns = jnp.clip(deg_out, 1.0) ** -0.5
    nd = jnp.clip(deg_in, 1.0) ** -0.5
    ns3, nd3 = _vec3(ns), _vec3(nd)

    srcr, dstr = _pad_edges(src, dst)
    zeros = jnp.zeros((_ZROW, D_IN), jnp.float32)

    feature = _fm_stage(input_feats, input_labels, p)
    hn0 = feature * ns[:, None]
    part0, = _segsum_sc([hn0], srcr, dstr, zeros)
    h, hn1_0, hn1_1, hn1_2, hn1_3 = _layer0_stage(feature, part0, nd3, ns3, p)
    parts1 = _segsum_sc([hn1_0, hn1_1, hn1_2, hn1_3], srcr, dstr, zeros)
    logits_pad = _layer1_stage(h, parts1, feature, nd3, p)
    return logits_pad[:, :N_CLASS]
